# Initial kernel scaffold; baseline (speedup 1.0000x reference)
import jax
import jax.numpy as jnp
from jax.experimental import pallas as pl


def kernel(x, input_pos, k_cache, v_cache):
    raise NotImplementedError("write your pallas kernel here")


# speedup vs baseline: 1.0614x; 1.0614x over previous
import functools

import jax
import jax.numpy as jnp
from jax import lax
from jax.experimental import pallas as pl
from jax.experimental.pallas import tpu as pltpu
from jax.experimental.pallas import tpu_sc as plsc

DIM = 2048
MAX_LEN = 8192
SEQ = 1024

NC = 2
NS = 16
L = 16
NW = NC * NS
CHUNK = DIM // NW


def _sc_body(x_hbm, pos_hbm, k_hbm, v_hbm, out_hbm, xv, kv, vv, ov, posv):
    wid = lax.axis_index("s") * NC + lax.axis_index("c")
    base = wid * CHUNK
    pltpu.sync_copy(pos_hbm, posv)
    pltpu.sync_copy(x_hbm.at[pl.ds(base, CHUNK)], xv)
    pltpu.sync_copy(k_hbm.at[pl.ds(base, CHUNK)], kv)
    pltpu.sync_copy(v_hbm.at[pl.ds(base, CHUNK)], vv)
    pv = posv[...]
    wrote = (pv == 0) | (pv <= -MAX_LEN)
    for j in range(CHUNK // L):
        sl = pl.ds(j * L, L)
        xs = xv[sl]
        ks = kv[sl]
        vs = vv[sl]
        new0 = jnp.maximum(xs * 2.0 + 1.0, 0.0) + ks
        ov[sl] = jnp.where(wrote, new0 + new0, ks + vs)
    pltpu.sync_copy(ov, out_hbm.at[pl.ds(base, CHUNK)])


_sc_row_update = functools.partial(
    pl.kernel,
    mesh=plsc.VectorSubcoreMesh(core_axis_name="c", subcore_axis_name="s"),
    out_type=jax.ShapeDtypeStruct((DIM,), jnp.float32),
    scratch_types=[
        pltpu.VMEM((CHUNK,), jnp.float32),
        pltpu.VMEM((CHUNK,), jnp.float32),
        pltpu.VMEM((CHUNK,), jnp.float32),
        pltpu.VMEM((CHUNK,), jnp.float32),
        pltpu.VMEM((L,), jnp.int32),
    ],
)(_sc_body)


def kernel(x, input_pos, k_cache, v_cache):
    pos16 = jnp.broadcast_to(input_pos, (L,))
    out = _sc_row_update(
        x.reshape(-1),
        pos16,
        k_cache.reshape(-1),
        v_cache.reshape(-1),
    )
    return out.reshape(1, 1, 1, DIM)


# speedup vs baseline: 5.7222x; 5.3910x over previous
import functools

import jax
import jax.numpy as jnp
from jax import lax
from jax.experimental import pallas as pl
from jax.experimental.pallas import tpu as pltpu
from jax.experimental.pallas import tpu_sc as plsc

DIM = 2048
MAX_LEN = 8192

NC = 2
NS = 16
L = 16
NW = NC * NS
CHUNK = DIM // NW


def _sc_body(x_hbm, pos_hbm, k_hbm, v_hbm, out_hbm, xv, kv, vv, ov, posv):
    wid = lax.axis_index("s") * NC + lax.axis_index("c")
    base = wid * CHUNK
    pltpu.sync_copy(pos_hbm, posv)
    pltpu.sync_copy(x_hbm.at[pl.ds(base, CHUNK)], xv)
    pltpu.sync_copy(k_hbm.at[pl.ds(base, CHUNK)], kv)
    pltpu.sync_copy(v_hbm.at[pl.ds(base, CHUNK)], vv)
    pv = posv[...]
    wrote = (pv == 0) | (pv <= -MAX_LEN)
    for j in range(CHUNK // L):
        sl = pl.ds(j * L, L)
        xs = xv[sl]
        ks = kv[sl]
        vs = vv[sl]
        new0 = jnp.maximum(xs * 2.0 + 1.0, 0.0) + ks
        ov[sl] = jnp.where(wrote, new0 + new0, ks + vs)
    pltpu.sync_copy(ov, out_hbm.at[pl.ds(base, CHUNK)])


_sc_row_update = functools.partial(
    pl.kernel,
    mesh=plsc.VectorSubcoreMesh(core_axis_name="c", subcore_axis_name="s"),
    out_type=jax.ShapeDtypeStruct((DIM,), jnp.float32),
    scratch_types=[
        pltpu.VMEM((CHUNK,), jnp.float32),
        pltpu.VMEM((CHUNK,), jnp.float32),
        pltpu.VMEM((CHUNK,), jnp.float32),
        pltpu.VMEM((CHUNK,), jnp.float32),
        pltpu.VMEM((L,), jnp.int32),
    ],
)(_sc_body)


def kernel(x, input_pos, k_cache, v_cache):
    x0 = lax.slice(x, (0, 0, 0, 0), (1, 1, 1, DIM)).reshape(DIM)
    k0 = lax.slice(k_cache, (0, 0, 0, 0), (1, 1, 1, DIM)).reshape(DIM)
    v0 = lax.slice(v_cache, (0, 0, 0, 0), (1, 1, 1, DIM)).reshape(DIM)
    pos16 = jnp.broadcast_to(input_pos, (L,))
    out = _sc_row_update(x0, pos16, k0, v0)
    return out.reshape(1, 1, 1, DIM)
